# baseline (speedup 1.0000x reference)
import functools

import jax
import jax.numpy as jnp
from jax import lax
from jax.experimental import pallas as pl
from jax.experimental.pallas import tpu as pltpu
from jax.experimental.pallas import tpu_sc as plsc

B = 4096
L = 200
D = 64
N_CORES = 2
N_SUBCORES = 16
N_WORKERS = N_CORES * N_SUBCORES
S_PER_W = B // N_WORKERS
HALF_L = L // 2
ROWS_PER_W = S_PER_W * 2


def _pool_body(xr_hbm, table_hbm, out_hbm, idx_v, rows_v, pooled_v, sem):
    w = lax.axis_index("s") * N_CORES + lax.axis_index("c")
    row_base = w * ROWS_PER_W
    s_base = w * S_PER_W

    pltpu.sync_copy(xr_hbm.at[pl.ds(row_base, ROWS_PER_W)], idx_v)

    def sample_body(i, carry):
        cp0 = pltpu.async_copy(
            table_hbm.at[idx_v.at[2 * i]], rows_v.at[pl.ds(0, HALF_L)], sem)
        cp1 = pltpu.async_copy(
            table_hbm.at[idx_v.at[2 * i + 1]], rows_v.at[pl.ds(HALF_L, HALF_L)],
            sem)
        cp0.wait()
        cp1.wait()

        def row_body(r, acc):
            return tuple(
                acc[c] + rows_v[r, pl.ds(16 * c, 16)] for c in range(4))

        acc = lax.fori_loop(
            0, L, row_body,
            tuple(jnp.zeros((16,), jnp.float32) for _ in range(4)))
        inv = jnp.float32(1.0 / L)
        for c in range(4):
            pooled_v[i, pl.ds(16 * c, 16)] = acc[c] * inv
        return carry

    lax.fori_loop(0, S_PER_W, sample_body, 0)
    pltpu.sync_copy(pooled_v, out_hbm.at[pl.ds(s_base, S_PER_W)])


def _pool(xr, table):
    mesh = plsc.VectorSubcoreMesh(core_axis_name="c", subcore_axis_name="s")
    kern = functools.partial(
        pl.kernel,
        mesh=mesh,
        compiler_params=pltpu.CompilerParams(use_tc_tiling_on_sc=False),
        out_type=jax.ShapeDtypeStruct((B, D), jnp.float32),
        scratch_types=[
            pltpu.VMEM((ROWS_PER_W, HALF_L), jnp.int32),
            pltpu.VMEM((L, D), jnp.float32),
            pltpu.VMEM((S_PER_W, D), jnp.float32),
            pltpu.SemaphoreType.DMA,
        ],
    )(_pool_body)
    return kern(xr, table)


def _mlp_body(h_ref, w1_ref, b1_ref, w2_ref, b2_ref, out_ref):
    h = h_ref[...]
    z = jnp.maximum(
        lax.dot(h, w1_ref[...], preferred_element_type=jnp.float32)
        + b1_ref[...], 0.0)
    out_ref[...] = (
        lax.dot(z, w2_ref[...], preferred_element_type=jnp.float32)
        + b2_ref[...])


def kernel(x, table, W1, b1, W2, b2):
    xr = x.reshape(B * 2, HALF_L)
    pooled = _pool(xr, table)

    w2p = jnp.pad(W2, ((0, 0), (0, 128 - W2.shape[1])))
    b2p = jnp.pad(b2, (0, 128 - b2.shape[0])).reshape(1, 128)
    outp = pl.pallas_call(
        _mlp_body,
        out_shape=jax.ShapeDtypeStruct((B, 128), jnp.float32),
    )(pooled, W1, b1.reshape(1, D), w2p, b2p)
    return outp[:, :W2.shape[1]]
